# baseline (speedup 1.0000x reference)
import functools

import jax
import jax.numpy as jnp
from jax import lax
from jax.experimental import pallas as pl
from jax.experimental.pallas import tpu as pltpu
from jax.experimental.pallas import tpu_sc as plsc

_NEG = -1e30


def _pad_to(n, g=512):
    return ((n + g - 1) // g) * g


def _bm_for(p):
    for c in (512, 256, 128):
        if p % c == 0:
            return c
    return p


def _bk_for(p):
    for c in (2048, 1024, 512, 256, 128):
        if p % c == 0:
            return c
    return p


def _score_body(x_ref, p_ref, o_ref, *, n, bm):
    i = pl.program_id(0)
    pv = p_ref[...]
    pn = jnp.sqrt(jnp.sum(pv * pv))
    s = jnp.tanh(jnp.dot(x_ref[...], pv, preferred_element_type=jnp.float32) / pn)
    rows = i * bm + lax.broadcasted_iota(jnp.int32, (bm, 1), 0)
    o_ref[...] = jnp.where(rows < n, s, _NEG)


def _score(x, p, n):
    P, C = x.shape
    bm = _bm_for(P)
    return pl.pallas_call(
        functools.partial(_score_body, n=n, bm=bm),
        grid=(P // bm,),
        in_specs=[
            pl.BlockSpec((bm, C), lambda i: (i, 0)),
            pl.BlockSpec((C, 1), lambda i: (0, 0)),
        ],
        out_specs=pl.BlockSpec((bm, 1), lambda i: (i, 0)),
        out_shape=jax.ShapeDtypeStruct((P, 1), jnp.float32),
    )(x, p.reshape(C, 1))


def _gather_b_body(perm_ref, a_hbm, o_ref, buf, sems, *, R, k, nsteps, P):
    i = pl.program_id(0)

    def _copy(step, slot, r):
        return pltpu.make_async_copy(
            a_hbm.at[pl.ds(perm_ref[step * R + r], 1), :],
            buf.at[slot, pl.ds(r, 1), :],
            sems.at[slot, r],
        )

    def fire(step, slot):
        for r in range(R):
            _copy(step, slot, r).start()

    @pl.when(i == 0)
    def _():
        fire(0, 0)

    par = i % 2

    @pl.when(jnp.logical_and(i + 1 < nsteps, par == 0))
    def _():
        fire(i + 1, 1)

    @pl.when(jnp.logical_and(i + 1 < nsteps, par == 1))
    def _():
        fire(i + 1, 0)

    def process(slot):
        for r in range(R):
            _copy(i, slot, r).wait()
        data = buf[slot]
        riota = lax.broadcasted_iota(jnp.int32, (R, 1), 0)
        pv = jnp.zeros((R, 1), jnp.int32)
        for r in range(R):
            pv = jnp.where(riota == r, perm_ref[i * R + r], pv)
        cols = lax.broadcasted_iota(jnp.int32, (R, P), 1)
        rows = i * R + lax.broadcasted_iota(jnp.int32, (R, P), 0)
        fixed = jnp.where(cols == pv, 1.0, data)
        fixed = jnp.where(rows < k, fixed, 0.0)
        o_ref[...] = fixed.astype(o_ref.dtype)

    @pl.when(par == 0)
    def _():
        process(0)

    @pl.when(par == 1)
    def _():
        process(1)


def _gather_b(a, perm, k, kp, dtype, rows_per_step=16):
    P = a.shape[1]
    R = rows_per_step
    nsteps = kp // R
    grid_spec = pltpu.PrefetchScalarGridSpec(
        num_scalar_prefetch=1,
        grid=(nsteps,),
        in_specs=[pl.BlockSpec(memory_space=pl.ANY)],
        out_specs=pl.BlockSpec((R, P), lambda i, perm_ref: (i, 0)),
        scratch_shapes=[
            pltpu.VMEM((2, R, P), jnp.float32),
            pltpu.SemaphoreType.DMA((2, R)),
        ],
    )
    return pl.pallas_call(
        functools.partial(_gather_b_body, R=R, k=k, nsteps=nsteps, P=P),
        grid_spec=grid_spec,
        out_shape=jax.ShapeDtypeStruct((kp, P), dtype),
    )(perm, a)


def _mm_nn_body(a_ref, b_ref, o_ref, acc_ref, *, nk, bm, bn):
    i, j, kk = pl.program_id(0), pl.program_id(1), pl.program_id(2)

    @pl.when(kk == 0)
    def _():
        acc_ref[...] = jnp.zeros_like(acc_ref)

    acc_ref[...] += jnp.dot(a_ref[...], b_ref[...],
                            preferred_element_type=jnp.float32)

    @pl.when(kk == nk - 1)
    def _():
        acc = acc_ref[...]
        rows = i * bm + lax.broadcasted_iota(jnp.int32, (bm, bn), 0)
        cols = j * bn + lax.broadcasted_iota(jnp.int32, (bm, bn), 1)
        o_ref[...] = jnp.where(rows == cols, 0.0, acc)


def _square_pooled(rp, cpt):
    Kp, P = rp.shape
    bm = bn = _bm_for(Kp)
    bk = _bk_for(P)
    nk = P // bk
    return pl.pallas_call(
        functools.partial(_mm_nn_body, nk=nk, bm=bm, bn=bn),
        grid=(Kp // bm, Kp // bn, nk),
        in_specs=[
            pl.BlockSpec((bm, bk), lambda i, j, kk: (i, kk)),
            pl.BlockSpec((bk, bn), lambda i, j, kk: (kk, j)),
        ],
        out_specs=pl.BlockSpec((bm, bn), lambda i, j, kk: (i, j)),
        out_shape=jax.ShapeDtypeStruct((Kp, Kp), jnp.float32),
        scratch_shapes=[pltpu.VMEM((bm, bn), jnp.float32)],
        compiler_params=pltpu.CompilerParams(
            dimension_semantics=("parallel", "parallel", "arbitrary")),
    )(rp, cpt)


def _transpose_body(a_ref, at_ref):
    at_ref[...] = a_ref[...].T


def _transpose(a):
    M, N = a.shape
    bm = _bm_for(N)
    bn = _bm_for(M)
    return pl.pallas_call(
        _transpose_body,
        grid=(N // bm, M // bn),
        in_specs=[pl.BlockSpec((bn, bm), lambda i, j: (j, i))],
        out_specs=pl.BlockSpec((bm, bn), lambda i, j: (i, j)),
        out_shape=jax.ShapeDtypeStruct((N, M), a.dtype),
        compiler_params=pltpu.CompilerParams(
            dimension_semantics=("parallel", "parallel")),
    )(a)


def _transpose_deg_body(a_ref, at_ref, deg_ref, acc_ref, *, nj):
    j = pl.program_id(1)
    t = a_ref[...].T

    @pl.when(j == 0)
    def _():
        acc_ref[...] = jnp.zeros_like(acc_ref)

    at_ref[...] = t
    acc_ref[...] += jnp.sum(t, axis=1, keepdims=True)

    @pl.when(j == nj - 1)
    def _():
        deg_ref[...] = acc_ref[...]


def _transpose_deg(a):
    Kp = a.shape[0]
    bm = bn = _bm_for(Kp)
    nj = Kp // bn
    return pl.pallas_call(
        functools.partial(_transpose_deg_body, nj=nj),
        grid=(Kp // bm, nj),
        in_specs=[pl.BlockSpec((bn, bm), lambda i, j: (j, i))],
        out_specs=[
            pl.BlockSpec((bm, bn), lambda i, j: (i, j)),
            pl.BlockSpec((bm, 1), lambda i, j: (i, 0)),
        ],
        out_shape=[
            jax.ShapeDtypeStruct((Kp, Kp), jnp.float32),
            jax.ShapeDtypeStruct((Kp, 1), jnp.float32),
        ],
        scratch_shapes=[pltpu.VMEM((bm, 1), jnp.float32)],
        compiler_params=pltpu.CompilerParams(
            dimension_semantics=("parallel", "arbitrary")),
    )(a)


def _xw_body(x_ref, w_ref, deg_ref, v_ref, o_ref, *, n, bm, kpool):
    i = pl.program_id(0)
    rows = i * bm + lax.broadcasted_iota(jnp.int32, (bm, 1), 0)
    xv = x_ref[...] * jnp.where(rows < kpool, v_ref[...], 0.0)
    y = jnp.dot(xv, w_ref[...], preferred_element_type=jnp.float32)
    dis = jnp.where(rows < n, lax.rsqrt(deg_ref[...] + 2.0), 0.0)
    o_ref[...] = dis * y


def _gcn_agg_body(at_ref, yk_ref, yr_ref, deg_ref, b_ref, o_ref, acc_ref,
                  *, nk, n, relu, bm):
    i, kk = pl.program_id(0), pl.program_id(1)

    @pl.when(kk == 0)
    def _():
        acc_ref[...] = jnp.zeros_like(acc_ref)

    acc_ref[...] += jnp.dot(at_ref[...], yk_ref[...],
                            preferred_element_type=jnp.float32)

    @pl.when(kk == nk - 1)
    def _():
        rows = i * bm + lax.broadcasted_iota(jnp.int32, (bm, 1), 0)
        dis = jnp.where(rows < n, lax.rsqrt(deg_ref[...] + 2.0), 0.0)
        out = dis * (acc_ref[...] + 2.0 * yr_ref[...]) + b_ref[...]
        if relu:
            out = jnp.maximum(out, 0.0)
        o_ref[...] = jnp.where(rows < n, out, 0.0)


def _gcn(x, at, deg, w, b, n, relu, vals=None, kpool=None):
    P, C = x.shape
    bm = _bm_for(P)
    if vals is None:
        vals = jnp.ones((P, 1), jnp.float32)
        kpool = n
    yp = pl.pallas_call(
        functools.partial(_xw_body, n=n, bm=bm, kpool=kpool),
        grid=(P // bm,),
        in_specs=[
            pl.BlockSpec((bm, C), lambda i: (i, 0)),
            pl.BlockSpec((C, C), lambda i: (0, 0)),
            pl.BlockSpec((bm, 1), lambda i: (i, 0)),
            pl.BlockSpec((bm, 1), lambda i: (i, 0)),
        ],
        out_specs=pl.BlockSpec((bm, C), lambda i: (i, 0)),
        out_shape=jax.ShapeDtypeStruct((P, C), jnp.float32),
    )(x, w, deg, vals)
    bk = _bk_for(P)
    nk = P // bk
    return pl.pallas_call(
        functools.partial(_gcn_agg_body, nk=nk, n=n, relu=relu, bm=bm),
        grid=(P // bm, nk),
        in_specs=[
            pl.BlockSpec((bm, bk), lambda i, kk: (i, kk)),
            pl.BlockSpec((bk, C), lambda i, kk: (kk, 0)),
            pl.BlockSpec((bm, C), lambda i, kk: (i, 0)),
            pl.BlockSpec((bm, 1), lambda i, kk: (i, 0)),
            pl.BlockSpec((1, C), lambda i, kk: (0, 0)),
        ],
        out_specs=pl.BlockSpec((bm, C), lambda i, kk: (i, 0)),
        out_shape=jax.ShapeDtypeStruct((P, C), jnp.float32),
        scratch_shapes=[pltpu.VMEM((bm, C), jnp.float32)],
        compiler_params=pltpu.CompilerParams(
            dimension_semantics=("parallel", "arbitrary")),
    )(at, yp, yp, deg, b.reshape(1, C))


def _sc_gather_rows(x, perm, kp):
    P, C = x.shape
    nw = 32
    while kp % nw != 0 or (kp // nw) % 8 != 0:
        nw //= 2
    b = kp // nw
    mesh = plsc.VectorSubcoreMesh(core_axis_name="c", subcore_axis_name="s")

    @functools.partial(
        pl.kernel, mesh=mesh,
        out_type=jax.ShapeDtypeStruct((kp, C), jnp.float32),
        scratch_types=[
            pltpu.VMEM((b,), jnp.int32),
            pltpu.VMEM((b, C), jnp.float32),
            pltpu.SemaphoreType.DMA,
        ],
    )
    def k(x_hbm, perm_hbm, out_hbm, idx_v, rows_v, sem):
        wid = lax.axis_index("s") * 2 + lax.axis_index("c")

        @pl.when(wid < nw)
        def _():
            base = wid * b
            pltpu.sync_copy(perm_hbm.at[pl.ds(base, b)], idx_v)
            pltpu.async_copy(x_hbm.at[idx_v], rows_v, sem).wait()
            pltpu.sync_copy(rows_v, out_hbm.at[pl.ds(base, b)])

    return k(x, perm)


def _pool_sel_body(perm_ref, x_ref, v_ref, o_ref, acc_ref, *, nk, k, bm, bk):
    i, kk = pl.program_id(0), pl.program_id(1)

    @pl.when(kk == 0)
    def _():
        acc_ref[...] = jnp.zeros_like(acc_ref)

    rows = i * bm + lax.broadcasted_iota(jnp.int32, (bm, bk), 0)
    cols = kk * bk + lax.broadcasted_iota(jnp.int32, (bm, bk), 1)
    onehot = jnp.where((perm_ref[...] == cols) & (rows < k), 1.0, 0.0)
    acc_ref[...] += jnp.dot(onehot, x_ref[...],
                            preferred_element_type=jnp.float32)

    @pl.when(kk == nk - 1)
    def _():
        o_ref[...] = acc_ref[...] * v_ref[...]


def _pool_x(x, perm, vals, k, kp):
    P, C = x.shape
    bm = _bm_for(kp)
    bk = _bm_for(P)
    nk = P // bk
    return pl.pallas_call(
        functools.partial(_pool_sel_body, nk=nk, k=k, bm=bm, bk=bk),
        grid=(kp // bm, nk),
        in_specs=[
            pl.BlockSpec((bm, 1), lambda i, kk: (i, 0)),
            pl.BlockSpec((bk, C), lambda i, kk: (kk, 0)),
            pl.BlockSpec((bm, 1), lambda i, kk: (i, 0)),
        ],
        out_specs=pl.BlockSpec((bm, C), lambda i, kk: (i, 0)),
        out_shape=jax.ShapeDtypeStruct((kp, C), jnp.float32),
        scratch_shapes=[pltpu.VMEM((bm, C), jnp.float32)],
        compiler_params=pltpu.CompilerParams(
            dimension_semantics=("parallel", "arbitrary")),
    )(perm.reshape(kp, 1), x, vals.reshape(kp, 1))


def _unpool_sel_body(perm_ref, xs_ref, res_ref, o_ref, acc_ref,
                     *, nk, k, bm, bk):
    i, kk = pl.program_id(0), pl.program_id(1)

    @pl.when(kk == 0)
    def _():
        acc_ref[...] = jnp.zeros_like(acc_ref)

    rows = i * bm + lax.broadcasted_iota(jnp.int32, (bm, bk), 0)
    cols = kk * bk + lax.broadcasted_iota(jnp.int32, (bm, bk), 1)
    onehot = jnp.where((perm_ref[...] == rows) & (cols < k), 1.0, 0.0)
    acc_ref[...] += jnp.dot(onehot, xs_ref[...],
                            preferred_element_type=jnp.float32)

    @pl.when(kk == nk - 1)
    def _():
        o_ref[...] = acc_ref[...] + res_ref[...]


def _unpool_add(xsmall, res, perm, k):
    kp, C = xsmall.shape
    P = res.shape[0]
    bm = _bm_for(P)
    bk = _bm_for(kp)
    nk = kp // bk
    return pl.pallas_call(
        functools.partial(_unpool_sel_body, nk=nk, k=k, bm=bm, bk=bk),
        grid=(P // bm, nk),
        in_specs=[
            pl.BlockSpec((1, bk), lambda i, kk: (0, kk)),
            pl.BlockSpec((bk, C), lambda i, kk: (kk, 0)),
            pl.BlockSpec((bm, C), lambda i, kk: (i, 0)),
        ],
        out_specs=pl.BlockSpec((bm, C), lambda i, kk: (i, 0)),
        out_shape=jax.ShapeDtypeStruct((P, C), jnp.float32),
        scratch_shapes=[pltpu.VMEM((bm, C), jnp.float32)],
        compiler_params=pltpu.CompilerParams(
            dimension_semantics=("parallel", "arbitrary")),
    )(perm.reshape(1, kp), xsmall, res)


def kernel(x, edge_index, batch, clinical, Wdown, bdown, pw, Wup, bup):
    N, C = x.shape
    depth = Wup.shape[0]
    P0 = _pad_to(N)
    xp = jnp.zeros((P0, C), jnp.float32).at[:N].set(x)
    src = edge_index[0]
    dst = edge_index[1]

    A0 = jnp.zeros((P0, P0), jnp.float32).at[src, dst].add(1.0)
    A0t, deg0 = _transpose_deg(A0)

    cx = _gcn(xp, A0t, deg0, Wdown[0], bdown[0], n=N, relu=True)
    xs = [cx]
    Ats = [A0t]
    degs = [deg0]
    ns = [N]
    perms = []
    ks = []
    cA, cAt, n = A0, A0t, N
    for lvl in range(1, depth + 1):
        k = -(-n // 2)
        kp = _pad_to(k)
        s = _score(cx, pw[lvl - 1], n)
        vals, perm = lax.top_k(s[:, 0], kp)
        dt = jnp.bfloat16 if lvl <= 2 else jnp.float32
        rp = _gather_b(cA, perm, k, kp, dt)
        cp = _gather_b(cAt, perm, k, kp, dt)
        A1 = _square_pooled(rp, _transpose(cp))
        A1t, deg1 = _transpose_deg(A1)
        cx = _sc_gather_rows(cx, perm, kp)
        cx = _gcn(cx, A1t, deg1, Wdown[lvl], bdown[lvl], n=k, relu=True,
                  vals=vals.reshape(kp, 1), kpool=k)
        perms.append(perm)
        ks.append(k)
        if lvl < depth:
            xs.append(cx)
            Ats.append(A1t)
            degs.append(deg1)
            ns.append(k)
        cA, cAt, n = A1, A1t, k

    for i in range(depth):
        j = depth - 1 - i
        xu = _unpool_add(cx, xs[j], perms[j], ks[j])
        cx = _gcn(xu, Ats[j], degs[j], Wup[i], bup[i], n=ns[j],
                  relu=(i < depth - 1))
    return cx[:N]


# speedup vs baseline: 1.4983x; 1.4983x over previous
import functools

import jax
import jax.numpy as jnp
from jax import lax
from jax.experimental import pallas as pl
from jax.experimental.pallas import tpu as pltpu
from jax.experimental.pallas import tpu_sc as plsc

_NEG = -1e30


def _pad_to(n, g=512):
    return ((n + g - 1) // g) * g


def _bm_for(p):
    for c in (512, 256, 128):
        if p % c == 0:
            return c
    return p


def _bk_for(p):
    for c in (2048, 1024, 512, 256, 128):
        if p % c == 0:
            return c
    return p


def _score_body(x_ref, p_ref, o_ref, *, n, bm):
    i = pl.program_id(0)
    pv = p_ref[...]
    pn = jnp.sqrt(jnp.sum(pv * pv))
    s = jnp.tanh(jnp.dot(x_ref[...], pv, preferred_element_type=jnp.float32) / pn)
    rows = i * bm + lax.broadcasted_iota(jnp.int32, (bm, 1), 0)
    o_ref[...] = jnp.where(rows < n, s, _NEG)


def _score(x, p, n):
    P, C = x.shape
    bm = _bm_for(P)
    return pl.pallas_call(
        functools.partial(_score_body, n=n, bm=bm),
        grid=(P // bm,),
        in_specs=[
            pl.BlockSpec((bm, C), lambda i: (i, 0)),
            pl.BlockSpec((C, 1), lambda i: (0, 0)),
        ],
        out_specs=pl.BlockSpec((bm, 1), lambda i: (i, 0)),
        out_shape=jax.ShapeDtypeStruct((P, 1), jnp.float32),
    )(x, p.reshape(C, 1))


def _gather_b_body(perm_ref, a_hbm, o_ref, buf, sems, *, R, k, nsteps, P):
    i = pl.program_id(0)

    def _copy(step, slot, r):
        return pltpu.make_async_copy(
            a_hbm.at[pl.ds(perm_ref[step * R + r], 1), :],
            buf.at[slot, pl.ds(r, 1), :],
            sems.at[slot, r],
        )

    def fire(step, slot):
        for r in range(R):
            _copy(step, slot, r).start()

    @pl.when(i == 0)
    def _():
        fire(0, 0)

    par = i % 2

    @pl.when(jnp.logical_and(i + 1 < nsteps, par == 0))
    def _():
        fire(i + 1, 1)

    @pl.when(jnp.logical_and(i + 1 < nsteps, par == 1))
    def _():
        fire(i + 1, 0)

    def process(slot):
        for r in range(R):
            _copy(i, slot, r).wait()
        data = buf[slot]
        riota = lax.broadcasted_iota(jnp.int32, (R, 1), 0)
        pv = jnp.zeros((R, 1), jnp.int32)
        for r in range(R):
            pv = jnp.where(riota == r, perm_ref[i * R + r], pv)
        cols = lax.broadcasted_iota(jnp.int32, (R, P), 1)
        rows = i * R + lax.broadcasted_iota(jnp.int32, (R, P), 0)
        fixed = jnp.where(cols == pv, 1.0, data)
        fixed = jnp.where(rows < k, fixed, 0.0)
        o_ref[...] = fixed.astype(o_ref.dtype)

    @pl.when(par == 0)
    def _():
        process(0)

    @pl.when(par == 1)
    def _():
        process(1)


def _gather_b(a, perm, k, kp, dtype, rows_per_step=32):
    P = a.shape[1]
    R = rows_per_step
    nsteps = kp // R
    grid_spec = pltpu.PrefetchScalarGridSpec(
        num_scalar_prefetch=1,
        grid=(nsteps,),
        in_specs=[pl.BlockSpec(memory_space=pl.ANY)],
        out_specs=pl.BlockSpec((R, P), lambda i, perm_ref: (i, 0)),
        scratch_shapes=[
            pltpu.VMEM((2, R, P), jnp.float32),
            pltpu.SemaphoreType.DMA((2, R)),
        ],
    )
    return pl.pallas_call(
        functools.partial(_gather_b_body, R=R, k=k, nsteps=nsteps, P=P),
        grid_spec=grid_spec,
        out_shape=jax.ShapeDtypeStruct((kp, P), dtype),
    )(perm, a)


def _mm_nn_body(a_ref, b_ref, o_ref, acc_ref, *, nk, bm, bn):
    i, j, kk = pl.program_id(0), pl.program_id(1), pl.program_id(2)

    @pl.when(kk == 0)
    def _():
        acc_ref[...] = jnp.zeros_like(acc_ref)

    acc_ref[...] += jnp.dot(a_ref[...], b_ref[...],
                            preferred_element_type=jnp.float32)

    @pl.when(kk == nk - 1)
    def _():
        acc = acc_ref[...]
        rows = i * bm + lax.broadcasted_iota(jnp.int32, (bm, bn), 0)
        cols = j * bn + lax.broadcasted_iota(jnp.int32, (bm, bn), 1)
        o_ref[...] = jnp.where(rows == cols, 0.0, acc)


def _square_pooled(rp, cpt):
    Kp, P = rp.shape
    bm = bn = 1024 if Kp % 1024 == 0 else _bm_for(Kp)
    bk = _bk_for(P)
    nk = P // bk
    return pl.pallas_call(
        functools.partial(_mm_nn_body, nk=nk, bm=bm, bn=bn),
        grid=(Kp // bm, Kp // bn, nk),
        in_specs=[
            pl.BlockSpec((bm, bk), lambda i, j, kk: (i, kk)),
            pl.BlockSpec((bk, bn), lambda i, j, kk: (kk, j)),
        ],
        out_specs=pl.BlockSpec((bm, bn), lambda i, j, kk: (i, j)),
        out_shape=jax.ShapeDtypeStruct((Kp, Kp), jnp.float32),
        scratch_shapes=[pltpu.VMEM((bm, bn), jnp.float32)],
        compiler_params=pltpu.CompilerParams(
            dimension_semantics=("parallel", "parallel", "arbitrary")),
    )(rp, cpt)


def _transpose_body(a_ref, at_ref):
    at_ref[...] = a_ref[...].T


def _transpose(a):
    M, N = a.shape
    bm = _bm_for(N)
    bn = _bm_for(M)
    return pl.pallas_call(
        _transpose_body,
        grid=(N // bm, M // bn),
        in_specs=[pl.BlockSpec((bn, bm), lambda i, j: (j, i))],
        out_specs=pl.BlockSpec((bm, bn), lambda i, j: (i, j)),
        out_shape=jax.ShapeDtypeStruct((N, M), a.dtype),
        compiler_params=pltpu.CompilerParams(
            dimension_semantics=("parallel", "parallel")),
    )(a)


def _transpose_deg_body(a_ref, at_ref, deg_ref, acc_ref, *, nj):
    j = pl.program_id(1)
    t = a_ref[...].T

    @pl.when(j == 0)
    def _():
        acc_ref[...] = jnp.zeros_like(acc_ref)

    at_ref[...] = t
    acc_ref[...] += jnp.sum(t, axis=1, keepdims=True)

    @pl.when(j == nj - 1)
    def _():
        deg_ref[...] = acc_ref[...]


def _transpose_deg(a):
    Kp = a.shape[0]
    bm = bn = _bm_for(Kp)
    nj = Kp // bn
    return pl.pallas_call(
        functools.partial(_transpose_deg_body, nj=nj),
        grid=(Kp // bm, nj),
        in_specs=[pl.BlockSpec((bn, bm), lambda i, j: (j, i))],
        out_specs=[
            pl.BlockSpec((bm, bn), lambda i, j: (i, j)),
            pl.BlockSpec((bm, 1), lambda i, j: (i, 0)),
        ],
        out_shape=[
            jax.ShapeDtypeStruct((Kp, Kp), jnp.float32),
            jax.ShapeDtypeStruct((Kp, 1), jnp.float32),
        ],
        scratch_shapes=[pltpu.VMEM((bm, 1), jnp.float32)],
        compiler_params=pltpu.CompilerParams(
            dimension_semantics=("parallel", "arbitrary")),
    )(a)


def _xw_body(x_ref, w_ref, deg_ref, v_ref, o_ref, *, n, bm, kpool):
    i = pl.program_id(0)
    rows = i * bm + lax.broadcasted_iota(jnp.int32, (bm, 1), 0)
    xv = x_ref[...] * jnp.where(rows < kpool, v_ref[...], 0.0)
    y = jnp.dot(xv, w_ref[...], preferred_element_type=jnp.float32)
    dis = jnp.where(rows < n, lax.rsqrt(deg_ref[...] + 2.0), 0.0)
    o_ref[...] = dis * y


def _gcn_agg_body(at_ref, yk_ref, yr_ref, deg_ref, b_ref, o_ref, acc_ref,
                  *, nk, n, relu, bm):
    i, kk = pl.program_id(0), pl.program_id(1)

    @pl.when(kk == 0)
    def _():
        acc_ref[...] = jnp.zeros_like(acc_ref)

    acc_ref[...] += jnp.dot(at_ref[...], yk_ref[...],
                            preferred_element_type=jnp.float32)

    @pl.when(kk == nk - 1)
    def _():
        rows = i * bm + lax.broadcasted_iota(jnp.int32, (bm, 1), 0)
        dis = jnp.where(rows < n, lax.rsqrt(deg_ref[...] + 2.0), 0.0)
        out = dis * (acc_ref[...] + 2.0 * yr_ref[...]) + b_ref[...]
        if relu:
            out = jnp.maximum(out, 0.0)
        o_ref[...] = jnp.where(rows < n, out, 0.0)


def _gcn(x, at, deg, w, b, n, relu, vals=None, kpool=None):
    P, C = x.shape
    bm = _bm_for(P)
    if vals is None:
        vals = jnp.ones((P, 1), jnp.float32)
        kpool = n
    yp = pl.pallas_call(
        functools.partial(_xw_body, n=n, bm=bm, kpool=kpool),
        grid=(P // bm,),
        in_specs=[
            pl.BlockSpec((bm, C), lambda i: (i, 0)),
            pl.BlockSpec((C, C), lambda i: (0, 0)),
            pl.BlockSpec((bm, 1), lambda i: (i, 0)),
            pl.BlockSpec((bm, 1), lambda i: (i, 0)),
        ],
        out_specs=pl.BlockSpec((bm, C), lambda i: (i, 0)),
        out_shape=jax.ShapeDtypeStruct((P, C), jnp.float32),
    )(x, w, deg, vals)
    bk = _bk_for(P)
    nk = P // bk
    return pl.pallas_call(
        functools.partial(_gcn_agg_body, nk=nk, n=n, relu=relu, bm=bm),
        grid=(P // bm, nk),
        in_specs=[
            pl.BlockSpec((bm, bk), lambda i, kk: (i, kk)),
            pl.BlockSpec((bk, C), lambda i, kk: (kk, 0)),
            pl.BlockSpec((bm, C), lambda i, kk: (i, 0)),
            pl.BlockSpec((bm, 1), lambda i, kk: (i, 0)),
            pl.BlockSpec((1, C), lambda i, kk: (0, 0)),
        ],
        out_specs=pl.BlockSpec((bm, C), lambda i, kk: (i, 0)),
        out_shape=jax.ShapeDtypeStruct((P, C), jnp.float32),
        scratch_shapes=[pltpu.VMEM((bm, C), jnp.float32)],
        compiler_params=pltpu.CompilerParams(
            dimension_semantics=("parallel", "arbitrary")),
    )(at, yp, yp, deg, b.reshape(1, C))


def _sc_gather_rows(x, perm, kp):
    P, C = x.shape
    nw = 32
    while kp % nw != 0 or (kp // nw) % 8 != 0:
        nw //= 2
    b = kp // nw
    mesh = plsc.VectorSubcoreMesh(core_axis_name="c", subcore_axis_name="s")

    @functools.partial(
        pl.kernel, mesh=mesh,
        out_type=jax.ShapeDtypeStruct((kp, C), jnp.float32),
        scratch_types=[
            pltpu.VMEM((b,), jnp.int32),
            pltpu.VMEM((b, C), jnp.float32),
            pltpu.SemaphoreType.DMA,
        ],
    )
    def k(x_hbm, perm_hbm, out_hbm, idx_v, rows_v, sem):
        wid = lax.axis_index("s") * 2 + lax.axis_index("c")

        @pl.when(wid < nw)
        def _():
            base = wid * b
            pltpu.sync_copy(perm_hbm.at[pl.ds(base, b)], idx_v)
            pltpu.async_copy(x_hbm.at[idx_v], rows_v, sem).wait()
            pltpu.sync_copy(rows_v, out_hbm.at[pl.ds(base, b)])

    return k(x, perm)


def _unpool_sel_body(perm_ref, xs_ref, res_ref, o_ref, acc_ref,
                     *, nk, k, bm, bk):
    i, kk = pl.program_id(0), pl.program_id(1)

    @pl.when(kk == 0)
    def _():
        acc_ref[...] = jnp.zeros_like(acc_ref)

    rows = i * bm + lax.broadcasted_iota(jnp.int32, (bm, bk), 0)
    cols = kk * bk + lax.broadcasted_iota(jnp.int32, (bm, bk), 1)
    onehot = jnp.where((perm_ref[...] == rows) & (cols < k), 1.0, 0.0)
    acc_ref[...] += jnp.dot(onehot, xs_ref[...],
                            preferred_element_type=jnp.float32)

    @pl.when(kk == nk - 1)
    def _():
        o_ref[...] = acc_ref[...] + res_ref[...]


def _unpool_add(xsmall, res, perm, k):
    kp, C = xsmall.shape
    P = res.shape[0]
    bm = _bm_for(P)
    bk = _bm_for(kp)
    nk = kp // bk
    return pl.pallas_call(
        functools.partial(_unpool_sel_body, nk=nk, k=k, bm=bm, bk=bk),
        grid=(P // bm, nk),
        in_specs=[
            pl.BlockSpec((1, bk), lambda i, kk: (0, kk)),
            pl.BlockSpec((bk, C), lambda i, kk: (kk, 0)),
            pl.BlockSpec((bm, C), lambda i, kk: (i, 0)),
        ],
        out_specs=pl.BlockSpec((bm, C), lambda i, kk: (i, 0)),
        out_shape=jax.ShapeDtypeStruct((P, C), jnp.float32),
        scratch_shapes=[pltpu.VMEM((bm, C), jnp.float32)],
        compiler_params=pltpu.CompilerParams(
            dimension_semantics=("parallel", "arbitrary")),
    )(perm.reshape(1, kp), xsmall, res)


def kernel(x, edge_index, batch, clinical, Wdown, bdown, pw, Wup, bup):
    N, C = x.shape
    depth = Wup.shape[0]
    P0 = _pad_to(N)
    xp = jnp.zeros((P0, C), jnp.float32).at[:N].set(x)
    src = edge_index[0]
    dst = edge_index[1]

    A0 = jnp.zeros((P0, P0), jnp.float32).at[src, dst].add(1.0)
    A0t, deg0 = _transpose_deg(A0)

    cx = _gcn(xp, A0t, deg0, Wdown[0], bdown[0], n=N, relu=True)
    xs = [cx]
    Ats = [A0t]
    degs = [deg0]
    ns = [N]
    perms = []
    ks = []
    cA, cAt, n = A0, A0t, N
    for lvl in range(1, depth + 1):
        k = -(-n // 2)
        kp = _pad_to(k)
        s = _score(cx, pw[lvl - 1], n)
        vals, perm = lax.top_k(s[:, 0], kp)
        dt = jnp.bfloat16 if lvl <= 2 else jnp.float32
        rp = _gather_b(cA, perm, k, kp, dt)
        cp = _gather_b(cAt, perm, k, kp, dt)
        A1 = _square_pooled(rp, _transpose(cp))
        A1t, deg1 = _transpose_deg(A1)
        cx = _sc_gather_rows(cx, perm, kp)
        cx = _gcn(cx, A1t, deg1, Wdown[lvl], bdown[lvl], n=k, relu=True,
                  vals=vals.reshape(kp, 1), kpool=k)
        perms.append(perm)
        ks.append(k)
        if lvl < depth:
            xs.append(cx)
            Ats.append(A1t)
            degs.append(deg1)
            ns.append(k)
        cA, cAt, n = A1, A1t, k

    for i in range(depth):
        j = depth - 1 - i
        xu = _unpool_add(cx, xs[j], perms[j], ks[j])
        cx = _gcn(xu, Ats[j], degs[j], Wup[i], bup[i], n=ns[j],
                  relu=(i < depth - 1))
    return cx[:N]
